# baseline (speedup 1.0000x reference)
import jax
import jax.numpy as jnp
from jax.experimental import pallas as pl

_BM = 1024


def _matmul_prelu_kernel(w_ref, adj_ref, seq_ref, out_ref):
    y = jnp.dot(adj_ref[...], seq_ref[...], preferred_element_type=jnp.float32)
    w = w_ref[0, 0]
    out_ref[...] = jnp.where(y >= 0, y, w * y)


def kernel(seq, adj_avg, prelu_weight):
    n, d = seq.shape
    w2d = prelu_weight.reshape(1, 1)
    grid = (n // _BM,)
    return pl.pallas_call(
        _matmul_prelu_kernel,
        grid=grid,
        in_specs=[
            pl.BlockSpec((1, 1), lambda i: (0, 0)),
            pl.BlockSpec((_BM, n), lambda i: (i, 0)),
            pl.BlockSpec((n, d), lambda i: (0, 0)),
        ],
        out_specs=pl.BlockSpec((_BM, d), lambda i: (i, 0)),
        out_shape=jax.ShapeDtypeStruct((n, d), jnp.float32),
    )(w2d, adj_avg, seq)


# speedup vs baseline: 1.0723x; 1.0723x over previous
import jax
import jax.numpy as jnp
from jax.experimental import pallas as pl

_BM = 512


def _matmul_prelu_kernel(w_ref, adj_ref, seq_ref, out_ref):
    y = jnp.dot(
        adj_ref[...].astype(jnp.bfloat16),
        seq_ref[...].astype(jnp.bfloat16),
        preferred_element_type=jnp.float32,
    )
    w = w_ref[0, 0]
    out_ref[...] = jnp.where(y >= 0, y, w * y)


def kernel(seq, adj_avg, prelu_weight):
    n, d = seq.shape
    w2d = prelu_weight.reshape(1, 1)
    grid = (n // _BM,)
    return pl.pallas_call(
        _matmul_prelu_kernel,
        grid=grid,
        in_specs=[
            pl.BlockSpec((1, 1), lambda i: (0, 0)),
            pl.BlockSpec((_BM, n), lambda i: (i, 0)),
            pl.BlockSpec((n, d), lambda i: (0, 0)),
        ],
        out_specs=pl.BlockSpec((_BM, d), lambda i: (i, 0)),
        out_shape=jax.ShapeDtypeStruct((n, d), jnp.float32),
    )(w2d, adj_avg, seq)
